# Initial kernel scaffold; baseline (speedup 1.0000x reference)
import jax
import jax.numpy as jnp
from jax.experimental import pallas as pl


def kernel(x, edge_index, W_l, W_r, b_l):
    raise NotImplementedError("write your pallas kernel here")


# speedup vs baseline: 3.8316x; 3.8316x over previous
import functools

import jax
import jax.numpy as jnp
from jax import lax
from jax.experimental import pallas as pl
from jax.experimental.pallas import tpu as pltpu
from jax.experimental.pallas import tpu_sc as plsc

N_NODES = 10000
N_EDGES = 320000
D_FEAT = 128
D_HALF = D_FEAT // 2
N_CLASSES = 40

NC = 2
NS = 16
E_PER_TILE = N_EDGES // NS
CHUNK = 80
N_CHUNKS = E_PER_TILE // CHUNK
N_PER_TILE = 632
N_PAD = NS * N_PER_TILE


def _sc_aggregate(x_lo, x_hi, src, dst, z_feat, z_deg, ones_h):
  mesh = plsc.VectorSubcoreMesh(core_axis_name="c", subcore_axis_name="s")

  @functools.partial(
      pl.kernel,
      out_type=[
          jax.ShapeDtypeStruct((NC, NS, N_PER_TILE, D_HALF), jnp.float32),
          jax.ShapeDtypeStruct((NS, N_PER_TILE), jnp.float32),
      ],
      mesh=mesh,
      compiler_params=pltpu.CompilerParams(use_tc_tiling_on_sc=False),
      scratch_types=[
          pltpu.VMEM((CHUNK,), jnp.int32),
          pltpu.VMEM((CHUNK,), jnp.int32),
          pltpu.VMEM((CHUNK, D_HALF), jnp.float32),
          pltpu.VMEM((CHUNK,), jnp.float32),
          pltpu.VMEM((N_PER_TILE, D_HALF), jnp.float32),
          pltpu.VMEM((N_PER_TILE,), jnp.float32),
          pltpu.VMEM_SHARED((N_PAD, D_HALF), jnp.float32),
          pltpu.VMEM_SHARED((N_PAD,), jnp.float32),
          pltpu.SemaphoreType.DMA,
      ],
  )
  def body(xlo_hbm, xhi_hbm, src_hbm, dst_hbm, zf_hbm, zd_hbm, ones_hbm,
           agg_out, deg_out,
           idx_s, idx_d, rows, ones_v, stg, stg_d, agg_sh, deg_sh, sem):
    c = lax.axis_index("c")
    s = lax.axis_index("s")
    base_n = s * N_PER_TILE

    pltpu.sync_copy(zf_hbm, stg)
    pltpu.sync_copy(zd_hbm, stg_d)
    pltpu.sync_copy(stg, agg_sh.at[pl.ds(base_n, N_PER_TILE)])
    pltpu.sync_copy(stg_d, deg_sh.at[pl.ds(base_n, N_PER_TILE)])
    pltpu.sync_copy(ones_hbm, ones_v)
    plsc.subcore_barrier()

    e0 = s * E_PER_TILE

    def edge_sweep(x_half, with_deg):
      def chunk_body(j, carry):
        e = e0 + j * CHUNK
        pltpu.sync_copy(src_hbm.at[pl.ds(e, CHUNK)], idx_s)
        pltpu.sync_copy(dst_hbm.at[pl.ds(e, CHUNK)], idx_d)
        pltpu.async_copy(x_half.at[idx_s], rows, sem).wait()
        pltpu.sync_copy(rows, agg_sh.at[idx_d], add=True)
        if with_deg:
          pltpu.sync_copy(ones_v, deg_sh.at[idx_d], add=True)
        return carry
      lax.fori_loop(0, N_CHUNKS, chunk_body, 0)

    @pl.when(c == 0)
    def _():
      edge_sweep(xlo_hbm, True)

    @pl.when(c == 1)
    def _():
      edge_sweep(xhi_hbm, False)

    plsc.subcore_barrier()

    pltpu.sync_copy(agg_sh.at[pl.ds(base_n, N_PER_TILE)], stg)
    pltpu.sync_copy(stg, agg_out.at[c, s])

    @pl.when(c == 0)
    def _():
      pltpu.sync_copy(deg_sh.at[pl.ds(base_n, N_PER_TILE)], stg_d)
      pltpu.sync_copy(stg_d, deg_out.at[s])

  return body(x_lo, x_hi, src, dst, z_feat, z_deg, ones_h)


ROWS_BLK = 1000


def _finalize_body(agg_ref, deg_ref, x_ref, wl_ref, wr_ref, b_ref, out_ref):
  a = jnp.concatenate([agg_ref[0], agg_ref[1]], axis=1)
  mean = a / jnp.maximum(deg_ref[...], 1.0)
  h = (jnp.dot(mean, wl_ref[...], preferred_element_type=jnp.float32)
       + jnp.dot(x_ref[...], wr_ref[...], preferred_element_type=jnp.float32)
       + b_ref[...])
  m = jnp.max(h, axis=1, keepdims=True)
  lse = jnp.log(jnp.sum(jnp.exp(h - m), axis=1, keepdims=True)) + m
  out_ref[...] = h - lse


def _finalize(agg, deg, x, wl_t, wr_t, b2d):
  grid = (N_NODES // ROWS_BLK,)
  return pl.pallas_call(
      _finalize_body,
      grid=grid,
      in_specs=[
          pl.BlockSpec((NC, ROWS_BLK, D_HALF), lambda i: (0, i, 0)),
          pl.BlockSpec((ROWS_BLK, 1), lambda i: (i, 0)),
          pl.BlockSpec((ROWS_BLK, D_FEAT), lambda i: (i, 0)),
          pl.BlockSpec((D_FEAT, N_CLASSES), lambda i: (0, 0)),
          pl.BlockSpec((D_FEAT, N_CLASSES), lambda i: (0, 0)),
          pl.BlockSpec((1, N_CLASSES), lambda i: (0, 0)),
      ],
      out_specs=pl.BlockSpec((ROWS_BLK, N_CLASSES), lambda i: (i, 0)),
      out_shape=jax.ShapeDtypeStruct((N_NODES, N_CLASSES), jnp.float32),
  )(agg, deg, x, wl_t, wr_t, b2d)


def kernel(x, edge_index, W_l, W_r, b_l):
  src = edge_index[0].astype(jnp.int32)
  dst = edge_index[1].astype(jnp.int32)
  x_lo = x[:, :D_HALF]
  x_hi = x[:, D_HALF:]
  z_feat = jnp.zeros((N_PER_TILE, D_HALF), jnp.float32)
  z_deg = jnp.zeros((N_PER_TILE,), jnp.float32)
  ones_h = jnp.ones((CHUNK,), jnp.float32)
  agg, deg = _sc_aggregate(x_lo, x_hi, src, dst, z_feat, z_deg, ones_h)
  agg = agg.reshape(NC, N_PAD, D_HALF)[:, :N_NODES]
  deg = deg.reshape(N_PAD, 1)[:N_NODES]
  return _finalize(agg, deg, x, W_l.T, W_r.T, b_l.reshape(1, N_CLASSES))


# speedup vs baseline: 7.8982x; 2.0613x over previous
import functools

import jax
import jax.numpy as jnp
from jax import lax
from jax.experimental import pallas as pl
from jax.experimental.pallas import tpu as pltpu
from jax.experimental.pallas import tpu_sc as plsc

N_NODES = 10000
N_EDGES = 320000
D_FEAT = 128
D_HALF = D_FEAT // 2
N_CLASSES = 40

NC = 2
NS = 16
CHUNK = 100
G = 5
NG = 40
N_CHUNKS = G * NG
E_PER_TILE = N_CHUNKS * CHUNK
E_PAD = NS * E_PER_TILE
N_PER_TILE = 632
N_PAD = NS * N_PER_TILE


def _sc_aggregate(x_lo, x_hi, src3, dst3, z_feat, z_deg, ones_h):
  mesh = plsc.VectorSubcoreMesh(core_axis_name="c", subcore_axis_name="s")

  @functools.partial(
      pl.kernel,
      out_type=[
          jax.ShapeDtypeStruct((NC, NS, N_PER_TILE, D_HALF), jnp.float32),
          jax.ShapeDtypeStruct((NS, N_PER_TILE), jnp.float32),
      ],
      mesh=mesh,
      compiler_params=pltpu.CompilerParams(use_tc_tiling_on_sc=False),
      scratch_types=[
          pltpu.VMEM((G, CHUNK), jnp.int32),
          pltpu.VMEM((G, CHUNK), jnp.int32),
          [pltpu.VMEM((CHUNK, D_HALF), jnp.float32) for _ in range(G)],
          pltpu.VMEM((CHUNK,), jnp.float32),
          pltpu.VMEM((N_PER_TILE, D_HALF), jnp.float32),
          pltpu.VMEM((N_PER_TILE,), jnp.float32),
          pltpu.VMEM_SHARED((N_PAD, D_HALF), jnp.float32),
          pltpu.VMEM_SHARED((N_PAD,), jnp.float32),
          [pltpu.SemaphoreType.DMA for _ in range(G)],
          pltpu.SemaphoreType.DMA,
          pltpu.SemaphoreType.DMA,
      ],
  )
  def body(xlo_hbm, xhi_hbm, src_hbm, dst_hbm, zf_hbm, zd_hbm, ones_hbm,
           agg_out, deg_out,
           idx_s, idx_d, rows, ones_v, stg, stg_d, agg_sh, deg_sh,
           sem_g, sem_s, sem_o):
    c = lax.axis_index("c")
    s = lax.axis_index("s")
    base_n = s * N_PER_TILE

    pltpu.sync_copy(zf_hbm, stg)
    pltpu.sync_copy(zd_hbm, stg_d)
    pltpu.sync_copy(stg, agg_sh.at[pl.ds(base_n, N_PER_TILE)])
    pltpu.sync_copy(stg_d, deg_sh.at[pl.ds(base_n, N_PER_TILE)])
    pltpu.sync_copy(ones_hbm, ones_v)
    plsc.subcore_barrier()

    def sweep(x_half, with_deg):
      def main_body(i, carry):
        pltpu.sync_copy(src_hbm.at[s, i], idx_s)
        pltpu.sync_copy(dst_hbm.at[s, i], idx_d)
        gd = [pltpu.async_copy(x_half.at[idx_s.at[b]], rows[b],
                               sem_g[b]) for b in range(G)]
        drains = []
        for b in range(G):
          gd[b].wait()
          drains.append(pltpu.async_copy(rows[b], agg_sh.at[idx_d.at[b]],
                                         sem_s, add=True))
          if with_deg:
            drains.append(pltpu.async_copy(ones_v, deg_sh.at[idx_d.at[b]],
                                           sem_o, add=True))
        for d in drains:
          d.wait()
        return carry

      lax.fori_loop(0, NG, main_body, 0)

    @pl.when(c == 0)
    def _():
      sweep(xlo_hbm, True)

    @pl.when(c == 1)
    def _():
      sweep(xhi_hbm, False)

    plsc.subcore_barrier()

    pltpu.sync_copy(agg_sh.at[pl.ds(base_n, N_PER_TILE)], agg_out.at[c, s])

    @pl.when(c == 0)
    def _():
      pltpu.sync_copy(deg_sh.at[pl.ds(base_n, N_PER_TILE)], deg_out.at[s])

  return body(x_lo, x_hi, src3, dst3, z_feat, z_deg, ones_h)


ROWS_BLK = 1000


def _finalize_body(agg_ref, deg_ref, x_ref, wl_ref, wr_ref, b_ref, out_ref):
  a = jnp.concatenate([agg_ref[0], agg_ref[1]], axis=1)
  mean = a / jnp.maximum(deg_ref[...], 1.0)
  h = (jnp.dot(mean, wl_ref[...], preferred_element_type=jnp.float32)
       + jnp.dot(x_ref[...], wr_ref[...], preferred_element_type=jnp.float32)
       + b_ref[...])
  m = jnp.max(h, axis=1, keepdims=True)
  lse = jnp.log(jnp.sum(jnp.exp(h - m), axis=1, keepdims=True)) + m
  out_ref[...] = h - lse


def _finalize(agg, deg, x, wl_t, wr_t, b2d):
  grid = (N_NODES // ROWS_BLK,)
  return pl.pallas_call(
      _finalize_body,
      grid=grid,
      in_specs=[
          pl.BlockSpec((NC, ROWS_BLK, D_HALF), lambda i: (0, i, 0)),
          pl.BlockSpec((ROWS_BLK, 1), lambda i: (i, 0)),
          pl.BlockSpec((ROWS_BLK, D_FEAT), lambda i: (i, 0)),
          pl.BlockSpec((D_FEAT, N_CLASSES), lambda i: (0, 0)),
          pl.BlockSpec((D_FEAT, N_CLASSES), lambda i: (0, 0)),
          pl.BlockSpec((1, N_CLASSES), lambda i: (0, 0)),
      ],
      out_specs=pl.BlockSpec((ROWS_BLK, N_CLASSES), lambda i: (i, 0)),
      out_shape=jax.ShapeDtypeStruct((N_NODES, N_CLASSES), jnp.float32),
  )(agg, deg, x, wl_t, wr_t, b2d)


def kernel(x, edge_index, W_l, W_r, b_l):
  src = edge_index[0].astype(jnp.int32)
  dst = edge_index[1].astype(jnp.int32)
  src3 = src.reshape(NS, NG, G, CHUNK)
  dst3 = dst.reshape(NS, NG, G, CHUNK)
  x_lo = x[:, :D_HALF]
  x_hi = x[:, D_HALF:]
  z_feat = jnp.zeros((N_PER_TILE, D_HALF), jnp.float32)
  z_deg = jnp.zeros((N_PER_TILE,), jnp.float32)
  ones_h = jnp.ones((CHUNK,), jnp.float32)
  agg, deg = _sc_aggregate(x_lo, x_hi, src3, dst3, z_feat, z_deg, ones_h)
  agg = agg.reshape(NC, N_PAD, D_HALF)[:, :N_NODES]
  deg = deg.reshape(N_PAD, 1)[:N_NODES]
  return _finalize(agg, deg, x, W_l.T, W_r.T, b_l.reshape(1, N_CLASSES))


# speedup vs baseline: 8.0221x; 1.0157x over previous
import functools

import jax
import jax.numpy as jnp
from jax import lax
from jax.experimental import pallas as pl
from jax.experimental.pallas import tpu as pltpu
from jax.experimental.pallas import tpu_sc as plsc

N_NODES = 10000
N_EDGES = 320000
D_FEAT = 128
D_HALF = D_FEAT // 2
N_CLASSES = 40

NC = 2
NS = 16
CHUNK = 100
G = 5
NG = 40
N_CHUNKS = G * NG
E_PER_TILE = N_CHUNKS * CHUNK
E_PAD = NS * E_PER_TILE
N_PER_TILE = 632
N_PAD = NS * N_PER_TILE


def _sc_aggregate(x_lo, x_hi, src3, dst3, z_feat, z_deg, ones_h):
  mesh = plsc.VectorSubcoreMesh(core_axis_name="c", subcore_axis_name="s")

  @functools.partial(
      pl.kernel,
      out_type=[
          jax.ShapeDtypeStruct((NC, NS, N_PER_TILE, D_HALF), jnp.float32),
          jax.ShapeDtypeStruct((NS, N_PER_TILE), jnp.float32),
      ],
      mesh=mesh,
      compiler_params=pltpu.CompilerParams(use_tc_tiling_on_sc=False),
      scratch_types=[
          pltpu.VMEM((G, CHUNK), jnp.int32),
          pltpu.VMEM((G, CHUNK), jnp.int32),
          [pltpu.VMEM((CHUNK, D_HALF), jnp.float32) for _ in range(G)],
          pltpu.VMEM((CHUNK,), jnp.float32),
          pltpu.VMEM((N_PER_TILE, D_HALF), jnp.float32),
          pltpu.VMEM((N_PER_TILE,), jnp.float32),
          pltpu.VMEM_SHARED((N_PAD, D_HALF), jnp.float32),
          pltpu.VMEM_SHARED((N_PAD,), jnp.float32),
          [pltpu.SemaphoreType.DMA for _ in range(G)],
          pltpu.SemaphoreType.DMA,
          pltpu.SemaphoreType.DMA,
      ],
  )
  def body(xlo_hbm, xhi_hbm, src_hbm, dst_hbm, zf_hbm, zd_hbm, ones_hbm,
           agg_out, deg_out,
           idx_s, idx_d, rows, ones_v, stg, stg_d, agg_sh, deg_sh,
           sem_g, sem_s, sem_o):
    c = lax.axis_index("c")
    s = lax.axis_index("s")
    base_n = s * N_PER_TILE

    pltpu.sync_copy(zf_hbm, stg)
    pltpu.sync_copy(zd_hbm, stg_d)
    pltpu.sync_copy(stg, agg_sh.at[pl.ds(base_n, N_PER_TILE)])
    pltpu.sync_copy(stg_d, deg_sh.at[pl.ds(base_n, N_PER_TILE)])
    pltpu.sync_copy(ones_hbm, ones_v)
    plsc.subcore_barrier()

    def sweep(x_half, with_deg):
      def main_body(i, carry):
        pltpu.sync_copy(src_hbm.at[s, i], idx_s)
        pltpu.sync_copy(dst_hbm.at[s, i], idx_d)
        gd = [pltpu.async_copy(x_half.at[idx_s.at[b]], rows[b],
                               sem_g[b]) for b in range(G)]
        drains = []
        for b in range(G):
          gd[b].wait()
          drains.append(pltpu.async_copy(rows[b], agg_sh.at[idx_d.at[b]],
                                         sem_s, add=True))
          if with_deg:
            drains.append(pltpu.async_copy(ones_v, deg_sh.at[idx_d.at[b]],
                                           sem_o, add=True))
        for d in drains:
          d.wait()
        return carry

      lax.fori_loop(0, NG, main_body, 0)

    @pl.when(c == 0)
    def _():
      sweep(xlo_hbm, True)

    @pl.when(c == 1)
    def _():
      sweep(xhi_hbm, False)

    plsc.subcore_barrier()

    pltpu.sync_copy(agg_sh.at[pl.ds(base_n, N_PER_TILE)], agg_out.at[c, s])

    @pl.when(c == 0)
    def _():
      pltpu.sync_copy(deg_sh.at[pl.ds(base_n, N_PER_TILE)], deg_out.at[s])

  return body(x_lo, x_hi, src3, dst3, z_feat, z_deg, ones_h)


ROWS_BLK = N_PER_TILE


def _finalize_body(agg_ref, deg_ref, x_ref, wl_ref, wr_ref, b_ref, out_ref):
  a = jnp.concatenate([agg_ref[0], agg_ref[1]], axis=1)
  mean = a / jnp.maximum(deg_ref[...], 1.0)
  h = (jnp.dot(mean, wl_ref[...], preferred_element_type=jnp.float32)
       + jnp.dot(x_ref[...], wr_ref[...], preferred_element_type=jnp.float32)
       + b_ref[...])
  m = jnp.max(h, axis=1, keepdims=True)
  lse = jnp.log(jnp.sum(jnp.exp(h - m), axis=1, keepdims=True)) + m
  out_ref[...] = h - lse


def _finalize(agg, deg, x, wl_t, wr_t, b2d):
  grid = (N_PAD // ROWS_BLK,)
  return pl.pallas_call(
      _finalize_body,
      grid=grid,
      in_specs=[
          pl.BlockSpec((NC, ROWS_BLK, D_HALF), lambda i: (0, i, 0)),
          pl.BlockSpec((ROWS_BLK, 1), lambda i: (i, 0)),
          pl.BlockSpec((ROWS_BLK, D_FEAT), lambda i: (i, 0)),
          pl.BlockSpec((D_FEAT, N_CLASSES), lambda i: (0, 0)),
          pl.BlockSpec((D_FEAT, N_CLASSES), lambda i: (0, 0)),
          pl.BlockSpec((1, N_CLASSES), lambda i: (0, 0)),
      ],
      out_specs=pl.BlockSpec((ROWS_BLK, N_CLASSES), lambda i: (i, 0)),
      out_shape=jax.ShapeDtypeStruct((N_PAD, N_CLASSES), jnp.float32),
  )(agg, deg, x, wl_t, wr_t, b2d)


def kernel(x, edge_index, W_l, W_r, b_l):
  src = edge_index[0].astype(jnp.int32)
  dst = edge_index[1].astype(jnp.int32)
  src3 = src.reshape(NS, NG, G, CHUNK)
  dst3 = dst.reshape(NS, NG, G, CHUNK)
  x_lo = x[:, :D_HALF]
  x_hi = x[:, D_HALF:]
  z_feat = jnp.zeros((N_PER_TILE, D_HALF), jnp.float32)
  z_deg = jnp.zeros((N_PER_TILE,), jnp.float32)
  ones_h = jnp.ones((CHUNK,), jnp.float32)
  agg, deg = _sc_aggregate(x_lo, x_hi, src3, dst3, z_feat, z_deg, ones_h)
  agg = agg.reshape(NC, N_PAD, D_HALF)
  deg = deg.reshape(N_PAD, 1)
  out = _finalize(agg, deg, x, W_l.T, W_r.T, b_l.reshape(1, N_CLASSES))
  return out[:N_NODES]


# speedup vs baseline: 9.2308x; 1.1507x over previous
import functools

import jax
import jax.numpy as jnp
from jax import lax
from jax.experimental import pallas as pl
from jax.experimental.pallas import tpu as pltpu
from jax.experimental.pallas import tpu_sc as plsc

N_NODES = 10000
N_EDGES = 320000
D_FEAT = 128
D_HALF = D_FEAT // 2
N_CLASSES = 40

NC = 2
NS = 16
CHUNK = 64
G = 10
NG = 32
N_CHUNKS = G * NG
E_PER_TILE = N_CHUNKS * CHUNK
E_PAD = NS * E_PER_TILE
N_PER_TILE = 632
N_PAD = NS * N_PER_TILE


def _sc_aggregate(x_lo, x_hi, src3, dst3, z_feat, z_deg, ones_h):
  mesh = plsc.VectorSubcoreMesh(core_axis_name="c", subcore_axis_name="s")

  @functools.partial(
      pl.kernel,
      out_type=[
          jax.ShapeDtypeStruct((NC, NS, N_PER_TILE, D_HALF), jnp.float32),
          jax.ShapeDtypeStruct((NS, N_PER_TILE), jnp.float32),
      ],
      mesh=mesh,
      compiler_params=pltpu.CompilerParams(use_tc_tiling_on_sc=False),
      scratch_types=[
          pltpu.VMEM((G, CHUNK), jnp.int32),
          pltpu.VMEM((G, CHUNK), jnp.int32),
          [pltpu.VMEM((CHUNK, D_HALF), jnp.float32) for _ in range(G)],
          pltpu.VMEM((CHUNK,), jnp.float32),
          pltpu.VMEM((N_PER_TILE, D_HALF), jnp.float32),
          pltpu.VMEM((N_PER_TILE,), jnp.float32),
          pltpu.VMEM_SHARED((N_PAD, D_HALF), jnp.float32),
          pltpu.VMEM_SHARED((N_PAD,), jnp.float32),
          [pltpu.SemaphoreType.DMA for _ in range(G)],
          pltpu.SemaphoreType.DMA,
          pltpu.SemaphoreType.DMA,
      ],
  )
  def body(xlo_hbm, xhi_hbm, src_hbm, dst_hbm, zf_hbm, zd_hbm, ones_hbm,
           agg_out, deg_out,
           idx_s, idx_d, rows, ones_v, stg, stg_d, agg_sh, deg_sh,
           sem_g, sem_s, sem_o):
    c = lax.axis_index("c")
    s = lax.axis_index("s")
    base_n = s * N_PER_TILE

    pltpu.sync_copy(zf_hbm, stg)
    pltpu.sync_copy(zd_hbm, stg_d)
    pltpu.sync_copy(stg, agg_sh.at[pl.ds(base_n, N_PER_TILE)])
    pltpu.sync_copy(stg_d, deg_sh.at[pl.ds(base_n, N_PER_TILE)])
    pltpu.sync_copy(ones_hbm, ones_v)
    plsc.subcore_barrier()

    def sweep(x_half, with_deg):
      def main_body(i, carry):
        pltpu.sync_copy(src_hbm.at[s, i], idx_s)
        pltpu.sync_copy(dst_hbm.at[s, i], idx_d)
        gd = [pltpu.async_copy(x_half.at[idx_s.at[b]], rows[b],
                               sem_g[b]) for b in range(G)]
        drains = []
        for b in range(G):
          gd[b].wait()
          drains.append(pltpu.async_copy(rows[b], agg_sh.at[idx_d.at[b]],
                                         sem_s, add=True))
          if with_deg:
            drains.append(pltpu.async_copy(ones_v, deg_sh.at[idx_d.at[b]],
                                           sem_o, add=True))
        for d in drains:
          d.wait()
        return carry

      lax.fori_loop(0, NG, main_body, 0)

    @pl.when(c == 0)
    def _():
      sweep(xlo_hbm, True)

    @pl.when(c == 1)
    def _():
      sweep(xhi_hbm, False)

    plsc.subcore_barrier()

    pltpu.sync_copy(agg_sh.at[pl.ds(base_n, N_PER_TILE)], agg_out.at[c, s])

    @pl.when(c == 0)
    def _():
      pltpu.sync_copy(deg_sh.at[pl.ds(base_n, N_PER_TILE)], deg_out.at[s])

  return body(x_lo, x_hi, src3, dst3, z_feat, z_deg, ones_h)


ROWS_BLK = N_PER_TILE


def _finalize_body(agg_ref, deg_ref, x_ref, wl_ref, wr_ref, b_ref, out_ref):
  a = jnp.concatenate([agg_ref[0], agg_ref[1]], axis=1)
  mean = a / jnp.maximum(deg_ref[...], 1.0)
  h = (jnp.dot(mean, wl_ref[...], preferred_element_type=jnp.float32)
       + jnp.dot(x_ref[...], wr_ref[...], preferred_element_type=jnp.float32)
       + b_ref[...])
  m = jnp.max(h, axis=1, keepdims=True)
  lse = jnp.log(jnp.sum(jnp.exp(h - m), axis=1, keepdims=True)) + m
  out_ref[...] = h - lse


def _finalize(agg, deg, x, wl_t, wr_t, b2d):
  grid = (N_PAD // ROWS_BLK,)
  return pl.pallas_call(
      _finalize_body,
      grid=grid,
      in_specs=[
          pl.BlockSpec((NC, ROWS_BLK, D_HALF), lambda i: (0, i, 0)),
          pl.BlockSpec((ROWS_BLK, 1), lambda i: (i, 0)),
          pl.BlockSpec((ROWS_BLK, D_FEAT), lambda i: (i, 0)),
          pl.BlockSpec((D_FEAT, N_CLASSES), lambda i: (0, 0)),
          pl.BlockSpec((D_FEAT, N_CLASSES), lambda i: (0, 0)),
          pl.BlockSpec((1, N_CLASSES), lambda i: (0, 0)),
      ],
      out_specs=pl.BlockSpec((ROWS_BLK, N_CLASSES), lambda i: (i, 0)),
      out_shape=jax.ShapeDtypeStruct((N_PAD, N_CLASSES), jnp.float32),
  )(agg, deg, x, wl_t, wr_t, b2d)


def kernel(x, edge_index, W_l, W_r, b_l):
  src = edge_index[0].astype(jnp.int32)
  dst = edge_index[1].astype(jnp.int32)
  pad = E_PAD - N_EDGES
  pad_src = jnp.arange(pad, dtype=jnp.int32) % N_NODES
  pad_dst = N_NODES + (jnp.arange(pad, dtype=jnp.int32) % (N_PAD - N_NODES))
  src3 = jnp.concatenate([src, pad_src]).reshape(NS, NG, G, CHUNK)
  dst3 = jnp.concatenate([dst, pad_dst]).reshape(NS, NG, G, CHUNK)
  x_lo = x[:, :D_HALF]
  x_hi = x[:, D_HALF:]
  z_feat = jnp.zeros((N_PER_TILE, D_HALF), jnp.float32)
  z_deg = jnp.zeros((N_PER_TILE,), jnp.float32)
  ones_h = jnp.ones((CHUNK,), jnp.float32)
  agg, deg = _sc_aggregate(x_lo, x_hi, src3, dst3, z_feat, z_deg, ones_h)
  agg = agg.reshape(NC, N_PAD, D_HALF)
  deg = deg.reshape(N_PAD, 1)
  out = _finalize(agg, deg, x, W_l.T, W_r.T, b_l.reshape(1, N_CLASSES))
  return out[:N_NODES]
